# Initial kernel scaffold; baseline (speedup 1.0000x reference)
import jax
import jax.numpy as jnp
from jax.experimental import pallas as pl


def kernel(x, edge_index, edge_type, basis, comp, root, bias1, w_rel, b_rel, w_root):
    raise NotImplementedError("write your pallas kernel here")


# speedup vs baseline: 12.9446x; 12.9446x over previous
import functools

import jax
import jax.numpy as jnp
from jax import lax
from jax.experimental import pallas as pl
from jax.experimental.pallas import tpu as pltpu
from jax.experimental.pallas import tpu_sc as plsc

N = 10000
E = 320000
F_IN = 128
H = 64
R = 16
NB = 30

NW = 32
CH = 512
SUB = 128
NSUB = CH // SUB
EPW = 10240
E_PAD = NW * EPW
NCHUNK = EPW // CH
N_PAD = 10240
RPT = 640
NQ = RPT // SUB
NKEY = N * R
CNT_SH = 160128


def _combine_body(comp_ref, basis_ref, out_ref):
    out_ref[...] = jnp.dot(comp_ref[...], basis_ref[...],
                           preferred_element_type=jnp.float32)


def _combine_basis(comp, basis2):
    return pl.pallas_call(
        _combine_body,
        out_shape=jax.ShapeDtypeStruct((R, F_IN * H), jnp.float32),
        name="rgcn_basis_combine",
    )(comp, basis2)


def _h_body(x_ref, wp_ref, root_ref, h_ref, xr_ref):
    xb = x_ref[...]
    h_ref[...] = jnp.dot(xb, wp_ref[0], preferred_element_type=jnp.float32)

    @pl.when(pl.program_id(1) == 0)
    def _():
        xr_ref[...] = jnp.dot(xb, root_ref[...],
                              preferred_element_type=jnp.float32)


def _h_kernel(x_p, w_pair, root):
    gi = N_PAD // RPT
    return pl.pallas_call(
        _h_body,
        grid=(gi, R // 2),
        in_specs=[
            pl.BlockSpec((RPT, F_IN), lambda i, t: (i, 0)),
            pl.BlockSpec((1, F_IN, 2 * H), lambda i, t: (t, 0, 0)),
            pl.BlockSpec((F_IN, H), lambda i, t: (0, 0)),
        ],
        out_specs=[
            pl.BlockSpec((RPT, 2 * H),
                         lambda i, t: (t * (N_PAD // RPT) + i, 0)),
            pl.BlockSpec((RPT, H), lambda i, t: (i, 0)),
        ],
        out_shape=[
            jax.ShapeDtypeStruct((R // 2 * N_PAD, 2 * H), jnp.float32),
            jax.ShapeDtypeStruct((N_PAD, H), jnp.float32),
        ],
        name="rgcn_messages_matmul",
    )(x_p, w_pair, root)


def _out1_body(a0_ref, a1_ref, xr_ref, b_ref, o_ref):
    val = a0_ref[...] + a1_ref[...] + xr_ref[...] + b_ref[...]
    row = lax.broadcasted_iota(jnp.int32, (N_PAD, H), 0)
    o_ref[...] = jnp.where(row < N, val, 0.0)


def _out1_kernel(aggp0, aggp1, xroot, bias1_2d):
    return pl.pallas_call(
        _out1_body,
        out_shape=jax.ShapeDtypeStruct((N_PAD, H), jnp.float32),
        name="rgcn_out1_assemble",
    )(aggp0, aggp1, xroot, bias1_2d)


def _out2_body(a0_ref, a1_ref, o1_ref, wrel_ref, wroot_ref, b_ref, o_ref):
    agg2 = a0_ref[...] + a1_ref[...]
    o_ref[...] = (jnp.dot(agg2, wrel_ref[...], preferred_element_type=jnp.float32)
                  + jnp.dot(o1_ref[...], wroot_ref[...],
                            preferred_element_type=jnp.float32)
                  + b_ref[...])


def _out2_kernel(agg2p0, agg2p1, out1_p, w_rel, w_root, brel_2d):
    bn = 2000
    grid = N // bn
    return pl.pallas_call(
        _out2_body,
        grid=(grid,),
        in_specs=[
            pl.BlockSpec((bn, H), lambda i: (i, 0)),
            pl.BlockSpec((bn, H), lambda i: (i, 0)),
            pl.BlockSpec((bn, H), lambda i: (i, 0)),
            pl.BlockSpec((H, H), lambda i: (0, 0)),
            pl.BlockSpec((H, H), lambda i: (0, 0)),
            pl.BlockSpec((1, H), lambda i: (0, 0)),
        ],
        out_specs=pl.BlockSpec((bn, H), lambda i: (i, 0)),
        out_shape=jax.ShapeDtypeStruct((N, H), jnp.float32),
        name="graphconv_out2_matmul",
    )(agg2p0, agg2p1, out1_p, w_rel, w_root, brel_2d)


_SC_MESH = plsc.VectorSubcoreMesh(core_axis_name="c", subcore_axis_name="s",
                                  num_cores=2, num_subcores=16)


def _zero_vmem_rows(ref, nrows):
    zero = jnp.zeros((16,), jnp.float32)

    def body(i, _):
        for k in range(H // 16):
            ref[i, pl.ds(16 * k, 16)] = zero
        return 0

    lax.fori_loop(0, nrows, body, 0)


def _fill_identity(zidx_v, base):
    for q in range(NQ):
        def ib(bb, _):
            zidx_v[q, pl.ds(16 * bb, 16)] = (
                lax.iota(jnp.int32, 16) + (base + q * SUB + 16 * bb))
            return 0
        lax.fori_loop(0, SUB // 16, ib, 0)


def _zero_agg(rows_v, agg_sh, zidx_v, sem):
    _zero_vmem_rows(rows_v, SUB)
    zs = [pltpu.async_copy(rows_v.at[pl.ds(0, SUB)],
                           agg_sh.at[zidx_v.at[q]], sem) for q in range(NQ)]
    for d in zs:
        d.wait()


def _export_agg(c, base, rows_v, agg_sh, zidx_v, agg0_hbm, agg1_hbm, sem):
    gts = [pltpu.async_copy(agg_sh.at[zidx_v.at[q]],
                            rows_v.at[pl.ds(q * SUB, SUB)], sem)
           for q in range(NSUB)]
    for d in gts:
        d.wait()

    @pl.when(c == 0)
    def _():
        pltpu.sync_copy(rows_v, agg0_hbm.at[pl.ds(base, CH)])

    @pl.when(c == 1)
    def _():
        pltpu.sync_copy(rows_v, agg1_hbm.at[pl.ds(base, CH)])

    pltpu.async_copy(agg_sh.at[zidx_v.at[NQ - 1]],
                     rows_v.at[pl.ds(0, SUB)], sem).wait()

    @pl.when(c == 0)
    def _():
        pltpu.sync_copy(rows_v.at[pl.ds(0, SUB)],
                        agg0_hbm.at[pl.ds(base + CH, SUB)])

    @pl.when(c == 1)
    def _():
        pltpu.sync_copy(rows_v.at[pl.ds(0, SUB)],
                        agg1_hbm.at[pl.ds(base + CH, SUB)])


def _count_body(key_hbm, cnt0_hbm, cnt1_hbm, key2_v, ones_v, zflat_v,
                cnt_sh, sem):
    c = lax.axis_index("c")
    s = lax.axis_index("s")
    wid = c * 16 + s

    one = jnp.full((16,), 1.0, jnp.float32)
    zero = jnp.zeros((16,), jnp.float32)
    for i in range(SUB // 16):
        ones_v[pl.ds(16 * i, 16)] = one

    def zb(i, _):
        zflat_v[pl.ds(16 * i, 16)] = zero
        return 0
    lax.fori_loop(0, 10016 // 16, zb, 0)

    pltpu.sync_copy(zflat_v.at[pl.ds(0, 10008)],
                    cnt_sh.at[pl.ds(s * 10008, 10008)])
    plsc.subcore_barrier()

    def chunk(ci, _):
        off = wid * EPW + ci * CH
        cps = [pltpu.async_copy(key_hbm.at[pl.ds(off + j * SUB, SUB)],
                                key2_v.at[j], sem) for j in range(NSUB)]
        for d in cps:
            d.wait()
        scs = [pltpu.async_copy(ones_v, cnt_sh.at[key2_v.at[j]], sem,
                                add=True) for j in range(NSUB)]
        for d in scs:
            d.wait()
        return 0

    lax.fori_loop(0, NCHUNK, chunk, 0)
    plsc.subcore_barrier()

    pltpu.sync_copy(cnt_sh.at[pl.ds(s * 10000, 10000)],
                    zflat_v.at[pl.ds(0, 10000)])

    @pl.when(c == 0)
    def _():
        pltpu.sync_copy(zflat_v.at[pl.ds(0, 10000)],
                        cnt0_hbm.at[pl.ds(s * 10000, 10000)])

    @pl.when(c == 1)
    def _():
        pltpu.sync_copy(zflat_v.at[pl.ds(0, 10000)],
                        cnt1_hbm.at[pl.ds(s * 10000, 10000)])


@functools.partial(
    pl.kernel,
    out_type=[jax.ShapeDtypeStruct((NKEY,), jnp.float32),
              jax.ShapeDtypeStruct((NKEY,), jnp.float32)],
    mesh=_SC_MESH,
    scratch_types=[
        pltpu.VMEM((NSUB, SUB), jnp.int32),
        pltpu.VMEM((SUB,), jnp.float32),
        pltpu.VMEM((10016,), jnp.float32),
        pltpu.VMEM_SHARED((CNT_SH,), jnp.float32),
        pltpu.SemaphoreType.DMA,
    ],
    name="sc_rgcn_key_bincount",
)
def _count_kernel(key_hbm, cnt0_hbm, cnt1_hbm, key2_v, ones_v, zflat_v,
                  cnt_sh, sem):
    _count_body(key_hbm, cnt0_hbm, cnt1_hbm, key2_v, ones_v, zflat_v,
                cnt_sh, sem)


def _mean_agg_body(h2_hbm, gidx_hbm, par_hbm, key_hbm, dst_hbm, norm_hbm,
                   agg0_hbm, agg1_hbm,
                   gidx2_v, par2_v, key2_v, dst2_v, nrm2_v, zidx_v, rows2_v,
                   rows_v, agg_sh, sem):
    c = lax.axis_index("c")
    s = lax.axis_index("s")
    wid = c * 16 + s
    base = s * RPT

    _fill_identity(zidx_v, base)
    _zero_agg(rows_v, agg_sh, zidx_v, sem)
    plsc.subcore_barrier()

    def chunk(ci, _):
        off = wid * EPW + ci * CH
        cps = []
        for j in range(NSUB):
            cps.append(pltpu.async_copy(
                gidx_hbm.at[pl.ds(off + j * SUB, SUB)], gidx2_v.at[j], sem))
            cps.append(pltpu.async_copy(
                par_hbm.at[pl.ds(off + j * SUB, SUB)], par2_v.at[j], sem))
            cps.append(pltpu.async_copy(
                key_hbm.at[pl.ds(off + j * SUB, SUB)], key2_v.at[j], sem))
            cps.append(pltpu.async_copy(
                dst_hbm.at[pl.ds(off + j * SUB, SUB)], dst2_v.at[j], sem))
        for d in cps:
            d.wait()
        gts = [pltpu.async_copy(norm_hbm.at[key2_v.at[j]], nrm2_v.at[j],
                                sem) for j in range(NSUB)]
        for d in gts:
            d.wait()

        for j in range(NSUB):
            pltpu.async_copy(h2_hbm.at[gidx2_v.at[j]], rows2_v, sem).wait()

            def sc_body(bb, _):
                nv = nrm2_v[j, pl.ds(16 * bb, 16)]
                pv = par2_v[j, pl.ds(16 * bb, 16)]
                for l in range(16):
                    rs = 16 * bb + l
                    nb = nv[l]
                    po = pv[l] * H
                    for k in range(H // 16):
                        rows_v[j * SUB + rs, pl.ds(16 * k, 16)] = (
                            rows2_v[rs, pl.ds(po + 16 * k, 16)] * nb)
                return 0
            lax.fori_loop(0, SUB // 16, sc_body, 0)

        scs = [pltpu.async_copy(rows_v.at[pl.ds(j * SUB, SUB)],
                                agg_sh.at[dst2_v.at[j]], sem, add=True)
               for j in range(NSUB)]
        for d in scs:
            d.wait()
        return 0

    lax.fori_loop(0, NCHUNK, chunk, 0)
    plsc.subcore_barrier()
    _export_agg(c, base, rows_v, agg_sh, zidx_v, agg0_hbm, agg1_hbm, sem)


@functools.partial(
    pl.kernel,
    out_type=[jax.ShapeDtypeStruct((N_PAD, H), jnp.float32),
              jax.ShapeDtypeStruct((N_PAD, H), jnp.float32)],
    mesh=_SC_MESH,
    scratch_types=[
        pltpu.VMEM((NSUB, SUB), jnp.int32),
        pltpu.VMEM((NSUB, SUB), jnp.int32),
        pltpu.VMEM((NSUB, SUB), jnp.int32),
        pltpu.VMEM((NSUB, SUB), jnp.int32),
        pltpu.VMEM((NSUB, SUB), jnp.float32),
        pltpu.VMEM((NQ, SUB), jnp.int32),
        pltpu.VMEM((SUB, 2 * H), jnp.float32),
        pltpu.VMEM((CH, H), jnp.float32),
        pltpu.VMEM_SHARED((N_PAD, H), jnp.float32),
        pltpu.SemaphoreType.DMA,
    ],
    name="sc_rgcn_mean_aggregate",
)
def _mean_agg_kernel(h2_hbm, gidx_hbm, par_hbm, key_hbm, dst_hbm, norm_hbm,
                     agg0_hbm, agg1_hbm,
                     gidx2_v, par2_v, key2_v, dst2_v, nrm2_v, zidx_v, rows2_v,
                     rows_v, agg_sh, sem):
    _mean_agg_body(h2_hbm, gidx_hbm, par_hbm, key_hbm, dst_hbm, norm_hbm,
                   agg0_hbm, agg1_hbm,
                   gidx2_v, par2_v, key2_v, dst2_v, nrm2_v, zidx_v, rows2_v,
                   rows_v, agg_sh, sem)


def _add_agg_body(x_hbm, src_hbm, spar_hbm, dst_hbm, agg0_hbm, agg1_hbm,
                  src2_v, spar2_v, dst2_v, zidx_v, rows2_v, rows_v, agg_sh,
                  sem):
    c = lax.axis_index("c")
    s = lax.axis_index("s")
    wid = c * 16 + s
    base = s * RPT

    _fill_identity(zidx_v, base)
    _zero_agg(rows_v, agg_sh, zidx_v, sem)
    plsc.subcore_barrier()

    def chunk(ci, _):
        off = wid * EPW + ci * CH
        cps = []
        for j in range(NSUB):
            cps.append(pltpu.async_copy(
                src_hbm.at[pl.ds(off + j * SUB, SUB)], src2_v.at[j], sem))
            cps.append(pltpu.async_copy(
                spar_hbm.at[pl.ds(off + j * SUB, SUB)], spar2_v.at[j], sem))
            cps.append(pltpu.async_copy(
                dst_hbm.at[pl.ds(off + j * SUB, SUB)], dst2_v.at[j], sem))
        for d in cps:
            d.wait()

        for j in range(NSUB):
            pltpu.async_copy(x_hbm.at[src2_v.at[j]], rows2_v, sem).wait()

            def ex_body(bb, _):
                pv = spar2_v[j, pl.ds(16 * bb, 16)]
                for l in range(16):
                    rs = 16 * bb + l
                    po = pv[l] * H
                    for k in range(H // 16):
                        rows_v[j * SUB + rs, pl.ds(16 * k, 16)] = (
                            rows2_v[rs, pl.ds(po + 16 * k, 16)])
                return 0
            lax.fori_loop(0, SUB // 16, ex_body, 0)

        scs = [pltpu.async_copy(rows_v.at[pl.ds(j * SUB, SUB)],
                                agg_sh.at[dst2_v.at[j]], sem, add=True)
               for j in range(NSUB)]
        for d in scs:
            d.wait()
        return 0

    lax.fori_loop(0, NCHUNK, chunk, 0)
    plsc.subcore_barrier()
    _export_agg(c, base, rows_v, agg_sh, zidx_v, agg0_hbm, agg1_hbm, sem)


@functools.partial(
    pl.kernel,
    out_type=[jax.ShapeDtypeStruct((N_PAD, H), jnp.float32),
              jax.ShapeDtypeStruct((N_PAD, H), jnp.float32)],
    mesh=_SC_MESH,
    scratch_types=[
        pltpu.VMEM((NSUB, SUB), jnp.int32),
        pltpu.VMEM((NSUB, SUB), jnp.int32),
        pltpu.VMEM((NSUB, SUB), jnp.int32),
        pltpu.VMEM((NQ, SUB), jnp.int32),
        pltpu.VMEM((SUB, 2 * H), jnp.float32),
        pltpu.VMEM((CH, H), jnp.float32),
        pltpu.VMEM_SHARED((N_PAD, H), jnp.float32),
        pltpu.SemaphoreType.DMA,
    ],
    name="sc_graphconv_add_aggregate",
)
def _add_agg_kernel(x_hbm, src_hbm, spar_hbm, dst_hbm, agg0_hbm, agg1_hbm,
                    src2_v, spar2_v, dst2_v, zidx_v, rows2_v, rows_v, agg_sh,
                    sem):
    _add_agg_body(x_hbm, src_hbm, spar_hbm, dst_hbm, agg0_hbm, agg1_hbm,
                  src2_v, spar2_v, dst2_v, zidx_v, rows2_v, rows_v, agg_sh,
                  sem)


def kernel(x, edge_index, edge_type, basis, comp, root, bias1, w_rel, b_rel,
           w_root):
    ei = edge_index.reshape(2, E)
    et = edge_type.reshape(E).astype(jnp.int32)
    src = ei[0].astype(jnp.int32)
    dst = ei[1].astype(jnp.int32)

    pad = E_PAD - E
    ar = jnp.arange(pad, dtype=jnp.int32)
    zpad = jnp.zeros((pad,), jnp.int32)
    gidx_p = jnp.concatenate([(et >> 1) * N_PAD + src, N + (ar % (N_PAD - N))])
    par_p = jnp.concatenate([et & 1, zpad])
    key_p = jnp.concatenate([dst * R + et, NKEY + (ar % SUB)])
    dst_p = jnp.concatenate([dst, ar % N])
    srcp_p = jnp.concatenate([src >> 1, (N >> 1) + (ar % ((N_PAD - N) >> 1))])
    spar_p = jnp.concatenate([src & 1, zpad])

    x_p = jnp.pad(x, ((0, N_PAD - N), (0, 0)))

    w2 = _combine_basis(comp, basis.reshape(NB, F_IN * H))
    w_pair = (w2.reshape(R // 2, 2, F_IN, H).transpose(0, 2, 1, 3)
              .reshape(R // 2, F_IN, 2 * H))
    h2, xroot = _h_kernel(x_p, w_pair, root)

    cnt0, cnt1 = _count_kernel(key_p)
    norm = 1.0 / jnp.maximum(cnt0 + cnt1, 1.0)
    norm_p = jnp.concatenate([norm, jnp.zeros((SUB,), jnp.float32)])

    agg0, agg1 = _mean_agg_kernel(h2, gidx_p, par_p, key_p, dst_p, norm_p)

    out1 = _out1_kernel(agg0, agg1, xroot, bias1.reshape(1, H))

    out1_2 = out1.reshape(N_PAD // 2, 2 * H)
    agg20, agg21 = _add_agg_kernel(out1_2, srcp_p, spar_p, dst_p)

    return _out2_kernel(agg20, agg21, out1, w_rel, w_root,
                        b_rel.reshape(1, H))


# speedup vs baseline: 15.8297x; 1.2229x over previous
import functools

import jax
import jax.numpy as jnp
from jax import lax
from jax.experimental import pallas as pl
from jax.experimental.pallas import tpu as pltpu
from jax.experimental.pallas import tpu_sc as plsc

N = 10000
E = 320000
F_IN = 128
H = 64
R = 16
NB = 30

NW = 32
CH = 512
SUB = 128
NSUB = CH // SUB
EPW = 10240
E_PAD = NW * EPW
NCHUNK = EPW // CH
N_PAD = 10240
RPT = 640
NQ = RPT // SUB
NKEY = N * R
CNT_SH = 160128


def _combine_body(comp_ref, basis_ref, out_ref):
    out_ref[...] = jnp.dot(comp_ref[...], basis_ref[...],
                           preferred_element_type=jnp.float32)


def _combine_basis(comp, basis2):
    return pl.pallas_call(
        _combine_body,
        out_shape=jax.ShapeDtypeStruct((R, F_IN * H), jnp.float32),
        name="rgcn_basis_combine",
    )(comp, basis2)


def _h_body(x_ref, wp_ref, root_ref, h_ref, xr_ref):
    xb = x_ref[...]
    h_ref[...] = jnp.dot(xb, wp_ref[0], preferred_element_type=jnp.float32)

    @pl.when(pl.program_id(1) == 0)
    def _():
        xr_ref[...] = jnp.dot(xb, root_ref[...],
                              preferred_element_type=jnp.float32)


def _h_kernel(x_p, w_pair, root):
    gi = N_PAD // RPT
    return pl.pallas_call(
        _h_body,
        grid=(gi, R // 2),
        in_specs=[
            pl.BlockSpec((RPT, F_IN), lambda i, t: (i, 0)),
            pl.BlockSpec((1, F_IN, 2 * H), lambda i, t: (t, 0, 0)),
            pl.BlockSpec((F_IN, H), lambda i, t: (0, 0)),
        ],
        out_specs=[
            pl.BlockSpec((RPT, 2 * H),
                         lambda i, t: (t * (N_PAD // RPT) + i, 0)),
            pl.BlockSpec((RPT, H), lambda i, t: (i, 0)),
        ],
        out_shape=[
            jax.ShapeDtypeStruct((R // 2 * N_PAD, 2 * H), jnp.float32),
            jax.ShapeDtypeStruct((N_PAD, H), jnp.float32),
        ],
        name="rgcn_messages_matmul",
    )(x_p, w_pair, root)


def _out1_body(a0_ref, a1_ref, xr_ref, b_ref, o_ref):
    a0 = a0_ref[...]
    a1 = a1_ref[...]
    val = (a0[:, :H] + a0[:, H:] + a1[:, :H] + a1[:, H:]
           + xr_ref[...] + b_ref[...])
    row = lax.broadcasted_iota(jnp.int32, (N_PAD, H), 0)
    o_ref[...] = jnp.where(row < N, val, 0.0)


def _out1_kernel(aggp0, aggp1, xroot, bias1_2d):
    return pl.pallas_call(
        _out1_body,
        out_shape=jax.ShapeDtypeStruct((N_PAD, H), jnp.float32),
        name="rgcn_out1_assemble",
    )(aggp0, aggp1, xroot, bias1_2d)


def _out2_body(a0_ref, a1_ref, o1_ref, wrel_ref, wroot_ref, b_ref, o_ref):
    a0 = a0_ref[...]
    a1 = a1_ref[...]
    agg2 = a0[:, :H] + a0[:, H:] + a1[:, :H] + a1[:, H:]
    o_ref[...] = (jnp.dot(agg2, wrel_ref[...], preferred_element_type=jnp.float32)
                  + jnp.dot(o1_ref[...], wroot_ref[...],
                            preferred_element_type=jnp.float32)
                  + b_ref[...])


def _out2_kernel(agg2p0, agg2p1, out1_p, w_rel, w_root, brel_2d):
    bn = 2000
    grid = N // bn
    return pl.pallas_call(
        _out2_body,
        grid=(grid,),
        in_specs=[
            pl.BlockSpec((bn, 2 * H), lambda i: (i, 0)),
            pl.BlockSpec((bn, 2 * H), lambda i: (i, 0)),
            pl.BlockSpec((bn, H), lambda i: (i, 0)),
            pl.BlockSpec((H, H), lambda i: (0, 0)),
            pl.BlockSpec((H, H), lambda i: (0, 0)),
            pl.BlockSpec((1, H), lambda i: (0, 0)),
        ],
        out_specs=pl.BlockSpec((bn, H), lambda i: (i, 0)),
        out_shape=jax.ShapeDtypeStruct((N, H), jnp.float32),
        name="graphconv_out2_matmul",
    )(agg2p0, agg2p1, out1_p, w_rel, w_root, brel_2d)


_SC_MESH = plsc.VectorSubcoreMesh(core_axis_name="c", subcore_axis_name="s",
                                  num_cores=2, num_subcores=16)


def _zero_vmem_rows(ref, nrows):
    zero = jnp.zeros((16,), jnp.float32)

    def body(i, _):
        for k in range(2 * H // 16):
            ref[i, pl.ds(16 * k, 16)] = zero
        return 0

    lax.fori_loop(0, nrows, body, 0)


def _fill_identity(zidx_v, base):
    for q in range(NQ):
        def ib(bb, _):
            zidx_v[q, pl.ds(16 * bb, 16)] = (
                lax.iota(jnp.int32, 16) + (base + q * SUB + 16 * bb))
            return 0
        lax.fori_loop(0, SUB // 16, ib, 0)


def _zero_agg(rows2_v, agg_sh, zidx_v, sem):
    _zero_vmem_rows(rows2_v, SUB)
    zs = [pltpu.async_copy(rows2_v, agg_sh.at[zidx_v.at[q]], sem)
          for q in range(NQ)]
    for d in zs:
        d.wait()
    pltpu.async_copy(agg_sh.at[zidx_v.at[0]], rows2_v, sem).wait()


def _export_agg(c, base, rows2_v, agg_sh, zidx_v, aggp_hbm, sem):
    for q in range(NQ):
        pltpu.async_copy(agg_sh.at[zidx_v.at[q]], rows2_v, sem).wait()
        pltpu.sync_copy(
            rows2_v, aggp_hbm.at[pl.ds(c * N_PAD + base + q * SUB, SUB)])


def _count_body(key_hbm, cntp_hbm, key2_v, ones_v, zflat_v,
                cnt_sh, sem):
    c = lax.axis_index("c")
    s = lax.axis_index("s")
    wid = c * 16 + s

    one = jnp.full((16,), 1.0, jnp.float32)
    zero = jnp.zeros((16,), jnp.float32)
    for i in range(SUB // 16):
        ones_v[pl.ds(16 * i, 16)] = one

    def zb(i, _):
        zflat_v[pl.ds(16 * i, 16)] = zero
        return 0
    lax.fori_loop(0, 10016 // 16, zb, 0)

    pltpu.sync_copy(zflat_v.at[pl.ds(0, 10008)],
                    cnt_sh.at[pl.ds(s * 10008, 10008)])
    plsc.subcore_barrier()

    def chunk(ci, _):
        off = wid * EPW + ci * CH
        cps = [pltpu.async_copy(key_hbm.at[pl.ds(off + j * SUB, SUB)],
                                key2_v.at[j], sem) for j in range(NSUB)]
        for d in cps:
            d.wait()
        scs = [pltpu.async_copy(ones_v, cnt_sh.at[key2_v.at[j]], sem,
                                add=True) for j in range(NSUB)]
        for d in scs:
            d.wait()
        return 0

    lax.fori_loop(0, NCHUNK, chunk, 0)
    plsc.subcore_barrier()

    pltpu.sync_copy(cnt_sh.at[pl.ds(s * 10000, 10000)],
                    zflat_v.at[pl.ds(0, 10000)])
    pltpu.sync_copy(zflat_v.at[pl.ds(0, 10000)],
                    cntp_hbm.at[pl.ds(c * NKEY + s * 10000, 10000)])


@functools.partial(
    pl.kernel,
    out_type=jax.ShapeDtypeStruct((2 * NKEY,), jnp.float32),
    mesh=_SC_MESH,
    scratch_types=[
        pltpu.VMEM((NSUB, SUB), jnp.int32),
        pltpu.VMEM((SUB,), jnp.float32),
        pltpu.VMEM((10016,), jnp.float32),
        pltpu.VMEM_SHARED((CNT_SH,), jnp.float32),
        pltpu.SemaphoreType.DMA,
    ],
    name="sc_rgcn_key_bincount",
)
def _count_kernel(key_hbm, cntp_hbm, key2_v, ones_v, zflat_v,
                  cnt_sh, sem):
    _count_body(key_hbm, cntp_hbm, key2_v, ones_v, zflat_v,
                cnt_sh, sem)


def _mean_agg_body(h2_hbm, gidx_hbm, par_hbm, key_hbm, dst_hbm, norm_hbm,
                   aggp_hbm,
                   gidx2_v, par2_v, key2_v, dst2_v, nrm2_v, zidx_v, rows2_v,
                   agg_sh, sem, sgr):
    c = lax.axis_index("c")
    s = lax.axis_index("s")
    wid = c * 16 + s
    base = s * RPT

    _fill_identity(zidx_v, base)
    _zero_agg(rows2_v, agg_sh, zidx_v, sem)
    plsc.subcore_barrier()

    def chunk(ci, _):
        off = wid * EPW + ci * CH
        cps = []
        for j in range(NSUB):
            cps.append(pltpu.async_copy(
                gidx_hbm.at[pl.ds(off + j * SUB, SUB)], gidx2_v.at[j], sem))
            cps.append(pltpu.async_copy(
                par_hbm.at[pl.ds(off + j * SUB, SUB)], par2_v.at[j], sem))
            cps.append(pltpu.async_copy(
                key_hbm.at[pl.ds(off + j * SUB, SUB)], key2_v.at[j], sem))
            cps.append(pltpu.async_copy(
                dst_hbm.at[pl.ds(off + j * SUB, SUB)], dst2_v.at[j], sem))
        for d in cps:
            d.wait()
        gts = [pltpu.async_copy(norm_hbm.at[key2_v.at[j]], nrm2_v.at[j],
                                sem) for j in range(NSUB)]
        for d in gts:
            d.wait()

        for j in range(NSUB):
            pltpu.async_copy(h2_hbm.at[gidx2_v.at[j]], rows2_v, sgr).wait()

            def sc_body(bb, _):
                nv = nrm2_v[j, pl.ds(16 * bb, 16)]
                pvf = par2_v[j, pl.ds(16 * bb, 16)].astype(jnp.float32)
                nv_r = nv * pvf
                nv_l = nv - nv_r
                for l in range(16):
                    rs = 16 * bb + l
                    nbl = nv_l[l]
                    nbr = nv_r[l]
                    for k in range(H // 16):
                        rows2_v[rs, pl.ds(16 * k, 16)] = (
                            rows2_v[rs, pl.ds(16 * k, 16)] * nbl)
                        rows2_v[rs, pl.ds(H + 16 * k, 16)] = (
                            rows2_v[rs, pl.ds(H + 16 * k, 16)] * nbr)
                return 0
            lax.fori_loop(0, SUB // 16, sc_body, 0)
            pltpu.async_copy(rows2_v, agg_sh.at[dst2_v.at[j]], sem,
                             add=True).wait()
        return 0

    lax.fori_loop(0, NCHUNK, chunk, 0)
    plsc.subcore_barrier()
    _export_agg(c, base, rows2_v, agg_sh, zidx_v, aggp_hbm, sem)


@functools.partial(
    pl.kernel,
    out_type=jax.ShapeDtypeStruct((2 * N_PAD, 2 * H), jnp.float32),
    mesh=_SC_MESH,
    scratch_types=[
        pltpu.VMEM((NSUB, SUB), jnp.int32),
        pltpu.VMEM((NSUB, SUB), jnp.int32),
        pltpu.VMEM((NSUB, SUB), jnp.int32),
        pltpu.VMEM((NSUB, SUB), jnp.int32),
        pltpu.VMEM((NSUB, SUB), jnp.float32),
        pltpu.VMEM((NQ, SUB), jnp.int32),
        pltpu.VMEM((SUB, 2 * H), jnp.float32),
        pltpu.VMEM_SHARED((N_PAD, 2 * H), jnp.float32),
        pltpu.SemaphoreType.DMA,
        pltpu.SemaphoreType.DMA,
    ],
    name="sc_rgcn_mean_aggregate",
)
def _mean_agg_kernel(h2_hbm, gidx_hbm, par_hbm, key_hbm, dst_hbm, norm_hbm,
                     aggp_hbm,
                     gidx2_v, par2_v, key2_v, dst2_v, nrm2_v, zidx_v, rows2_v,
                     agg_sh, sem, sgr):
    _mean_agg_body(h2_hbm, gidx_hbm, par_hbm, key_hbm, dst_hbm, norm_hbm,
                   aggp_hbm,
                   gidx2_v, par2_v, key2_v, dst2_v, nrm2_v, zidx_v, rows2_v,
                   agg_sh, sem, sgr)


def _add_agg_body(x_hbm, src_hbm, spar_hbm, dst_hbm, aggp_hbm,
                  src2_v, spar2_v, dst2_v, zidx_v, rows2_v, agg_sh,
                  sem, sgr):
    c = lax.axis_index("c")
    s = lax.axis_index("s")
    wid = c * 16 + s
    base = s * RPT

    _fill_identity(zidx_v, base)
    _zero_agg(rows2_v, agg_sh, zidx_v, sem)
    plsc.subcore_barrier()

    def chunk(ci, _):
        off = wid * EPW + ci * CH
        cps = []
        for j in range(NSUB):
            cps.append(pltpu.async_copy(
                src_hbm.at[pl.ds(off + j * SUB, SUB)], src2_v.at[j], sem))
            cps.append(pltpu.async_copy(
                spar_hbm.at[pl.ds(off + j * SUB, SUB)], spar2_v.at[j], sem))
            cps.append(pltpu.async_copy(
                dst_hbm.at[pl.ds(off + j * SUB, SUB)], dst2_v.at[j], sem))
        for d in cps:
            d.wait()

        for j in range(NSUB):
            pltpu.async_copy(x_hbm.at[src2_v.at[j]], rows2_v, sgr).wait()

            def ex_body(bb, _):
                pvf = spar2_v[j, pl.ds(16 * bb, 16)].astype(jnp.float32)
                one = jnp.full((16,), 1.0, jnp.float32)
                for l in range(16):
                    rs = 16 * bb + l
                    nbr = pvf[l]
                    nbl = one[l] - nbr
                    for k in range(H // 16):
                        rows2_v[rs, pl.ds(16 * k, 16)] = (
                            rows2_v[rs, pl.ds(16 * k, 16)] * nbl)
                        rows2_v[rs, pl.ds(H + 16 * k, 16)] = (
                            rows2_v[rs, pl.ds(H + 16 * k, 16)] * nbr)
                return 0
            lax.fori_loop(0, SUB // 16, ex_body, 0)
            pltpu.async_copy(rows2_v, agg_sh.at[dst2_v.at[j]], sem,
                             add=True).wait()
        return 0

    lax.fori_loop(0, NCHUNK, chunk, 0)
    plsc.subcore_barrier()
    _export_agg(c, base, rows2_v, agg_sh, zidx_v, aggp_hbm, sem)


@functools.partial(
    pl.kernel,
    out_type=jax.ShapeDtypeStruct((2 * N_PAD, 2 * H), jnp.float32),
    mesh=_SC_MESH,
    scratch_types=[
        pltpu.VMEM((NSUB, SUB), jnp.int32),
        pltpu.VMEM((NSUB, SUB), jnp.int32),
        pltpu.VMEM((NSUB, SUB), jnp.int32),
        pltpu.VMEM((NQ, SUB), jnp.int32),
        pltpu.VMEM((SUB, 2 * H), jnp.float32),
        pltpu.VMEM_SHARED((N_PAD, 2 * H), jnp.float32),
        pltpu.SemaphoreType.DMA,
        pltpu.SemaphoreType.DMA,
    ],
    name="sc_graphconv_add_aggregate",
)
def _add_agg_kernel(x_hbm, src_hbm, spar_hbm, dst_hbm, aggp_hbm,
                    src2_v, spar2_v, dst2_v, zidx_v, rows2_v, agg_sh,
                    sem, sgr):
    _add_agg_body(x_hbm, src_hbm, spar_hbm, dst_hbm, aggp_hbm,
                  src2_v, spar2_v, dst2_v, zidx_v, rows2_v, agg_sh,
                  sem, sgr)


def kernel(x, edge_index, edge_type, basis, comp, root, bias1, w_rel, b_rel,
           w_root):
    ei = edge_index.reshape(2, E)
    et = edge_type.reshape(E).astype(jnp.int32)
    src = ei[0].astype(jnp.int32)
    dst = ei[1].astype(jnp.int32)

    pad = E_PAD - E
    ar = jnp.arange(pad, dtype=jnp.int32)
    zpad = jnp.zeros((pad,), jnp.int32)
    gidx_p = jnp.concatenate([(et >> 1) * N_PAD + src, N + (ar % (N_PAD - N))])
    par_p = jnp.concatenate([et & 1, zpad])
    key_p = jnp.concatenate([dst * R + et, NKEY + (ar % SUB)])
    dst_p = jnp.concatenate([dst, ar % N])
    srcp_p = jnp.concatenate([src >> 1, (N >> 1) + (ar % ((N_PAD - N) >> 1))])
    spar_p = jnp.concatenate([src & 1, zpad])

    x_p = jnp.pad(x, ((0, N_PAD - N), (0, 0)))

    w2 = _combine_basis(comp, basis.reshape(NB, F_IN * H))
    w_pair = (w2.reshape(R // 2, 2, F_IN, H).transpose(0, 2, 1, 3)
              .reshape(R // 2, F_IN, 2 * H))
    h2, xroot = _h_kernel(x_p, w_pair, root)

    cntp = _count_kernel(key_p)
    norm = 1.0 / jnp.maximum(cntp[:NKEY] + cntp[NKEY:], 1.0)
    norm_p = jnp.concatenate([norm, jnp.zeros((SUB,), jnp.float32)])

    aggp = _mean_agg_kernel(h2, gidx_p, par_p, key_p, dst_p, norm_p)

    out1 = _out1_kernel(aggp[:N_PAD], aggp[N_PAD:], xroot,
                        bias1.reshape(1, H))

    out1_2 = out1.reshape(N_PAD // 2, 2 * H)
    agg2p = _add_agg_kernel(out1_2, srcp_p, spar_p, dst_p)

    return _out2_kernel(agg2p[:N_PAD], agg2p[N_PAD:], out1, w_rel, w_root,
                        b_rel.reshape(1, H))
